# Initial kernel scaffold; baseline (speedup 1.0000x reference)
import functools

import jax
import jax.numpy as jnp
from jax import lax
from jax.experimental import pallas as pl
from jax.experimental.pallas import tpu as pltpu
from jax.experimental.pallas import tpu_sc as plsc

H, W = 512, 512
HW = H * W
B = 8
N = 131072
NREG = 8
R = HW // NREG
NTASK = B * NREG
CHUNK = 4096
L = 16

_mesh = plsc.VectorSubcoreMesh(core_axis_name="c", subcore_axis_name="s")


@functools.partial(
    pl.kernel,
    mesh=_mesh,
    out_type=[
        jax.ShapeDtypeStruct((B, NREG, R), jnp.float32),
        jax.ShapeDtypeStruct((B, 3, NREG, 256, 128), jnp.float32),
        jax.ShapeDtypeStruct((B, NREG, R), jnp.int32),
    ],
    scratch_types=[
        pltpu.VMEM((CHUNK,), jnp.float32),
        pltpu.VMEM((CHUNK,), jnp.float32),
        pltpu.VMEM((CHUNK,), jnp.float32),
        pltpu.VMEM((R,), jnp.float32),
        pltpu.VMEM((R,), jnp.int32),
        pltpu.VMEM((64,), jnp.int32),
        pltpu.VMEM((256, 128), jnp.int32),
        pltpu.VMEM((256, 128), jnp.float32),
        pltpu.SemaphoreType.DMA,
    ],
)
def _sc_project(points_hbm, colors_hbm, depth_hbm, img_hbm, idx_hbm,
                xb, yb, zb, depth_v, id_v, tmp_v, idxstage, cstage, sem):
    wid = lax.axis_index("s") * 2 + lax.axis_index("c")
    lane = lax.iota(jnp.int32, L)
    inf16 = jnp.full((L,), jnp.inf, dtype=jnp.float32)
    n16 = jnp.full((L,), N, dtype=jnp.int32)

    for k in range(NTASK // 32):
        t = wid + 32 * k
        b = t >> 3
        q = t & 7

        def init_body(i, _):
            depth_v[pl.ds(i * L, L)] = inf16
            id_v[pl.ds(i * L, L)] = n16
            return 0
        lax.fori_loop(0, R // L, init_body, 0)

        def chunk_body(c, _):
            off = c * CHUNK
            pltpu.sync_copy(points_hbm.at[b, 0, pl.ds(off, CHUNK)], xb)
            pltpu.sync_copy(points_hbm.at[b, 1, pl.ds(off, CHUNK)], yb)
            pltpu.sync_copy(points_hbm.at[b, 2, pl.ds(off, CHUNK)], zb)

            def vec_body(j, _):
                s = pl.ds(j * L, L)
                x = xb[s]
                y = yb[s]
                z = zb[s]
                u = jnp.minimum(
                    jnp.maximum((x * jnp.float32(W)).astype(jnp.int32), 0),
                    W - 1)
                v = jnp.minimum(
                    jnp.maximum((y * jnp.float32(H)).astype(jnp.int32), 0),
                    H - 1)
                pix = (v << 9) | u
                in_reg = (pix >> 15) == q
                local = pix & (R - 1)
                slot = local & 63
                ids = (off + j * L) + lane
                d0 = plsc.load_gather(depth_v, [local])
                want = in_reg & (z < d0)

                def round_body(m):
                    plsc.store_scatter(tmp_v, [slot], lane, mask=m)
                    winner = plsc.load_gather(tmp_v, [slot])
                    lead = m & (winner == lane)
                    plsc.store_scatter(depth_v, [local], z, mask=lead)
                    plsc.store_scatter(id_v, [local], ids, mask=lead)
                    rest = m & jnp.logical_not(lead)
                    d1 = plsc.load_gather(depth_v, [local])
                    i1 = plsc.load_gather(id_v, [local])
                    return rest & ((z < d1) | ((z == d1) & (ids < i1)))

                lax.while_loop(jnp.any, round_body, want)
                return 0
            lax.fori_loop(0, CHUNK // L, vec_body, 0)
            return 0
        lax.fori_loop(0, N // CHUNK, chunk_body, 0)

        def fin_body(i, _):
            s = pl.ds(i * L, L)
            idv = id_v[s]
            dv = depth_v[s]
            valid = idv < N
            id_v[s] = jnp.where(valid, idv, -1)
            depth_v[s] = jnp.where(valid, dv, 0.0)
            row = i >> 3
            col = (i & 7) * L
            idxstage[row, pl.ds(col, L)] = jnp.where(valid, idv, 0)
            return 0
        lax.fori_loop(0, R // L, fin_body, 0)

        pltpu.sync_copy(depth_v, depth_hbm.at[b, q])
        pltpu.sync_copy(id_v, idx_hbm.at[b, q])

        for ch in range(3):
            pltpu.async_copy(colors_hbm.at[b, ch].at[idxstage], cstage,
                             sem).wait()

            def mask_body(i, _):
                row = i >> 3
                col = (i & 7) * L
                cv = cstage[row, pl.ds(col, L)]
                idv = id_v[pl.ds(i * L, L)]
                cstage[row, pl.ds(col, L)] = jnp.where(idv >= 0, cv, 0.0)
                return 0
            lax.fori_loop(0, R // L, mask_body, 0)
            pltpu.sync_copy(cstage, img_hbm.at[b, ch, q])


def kernel(points, colors):
    depth, img, index = _sc_project(points, colors)
    return (depth.reshape(B, H, W),
            img.reshape(B, 3, H, W),
            index.reshape(B, H, W))


# speedup vs baseline: 14.1836x; 14.1836x over previous
import functools

import jax
import jax.numpy as jnp
from jax import lax
from jax.experimental import pallas as pl
from jax.experimental.pallas import tpu as pltpu
from jax.experimental.pallas import tpu_sc as plsc

H, W = 512, 512
HW = H * W
B = 8
N = 131072
NREG = 8
R = HW // NREG
NTASK = B * NREG
CHUNK = 4096
L = 16
SUBC = 2048
ZSLOT = 3 * B * N

_mesh = plsc.VectorSubcoreMesh(core_axis_name="c", subcore_axis_name="s")


@functools.partial(
    pl.kernel,
    mesh=_mesh,
    out_type=[
        jax.ShapeDtypeStruct((B * NREG * R,), jnp.float32),
        jax.ShapeDtypeStruct((B * 3 * NREG * R,), jnp.float32),
        jax.ShapeDtypeStruct((B * NREG * R,), jnp.int32),
    ],
    scratch_types=[
        pltpu.VMEM((2 * CHUNK,), jnp.float32),
        pltpu.VMEM((2 * CHUNK,), jnp.float32),
        pltpu.VMEM((2 * CHUNK,), jnp.float32),
        pltpu.VMEM((R,), jnp.float32),
        pltpu.VMEM((R,), jnp.int32),
        pltpu.VMEM((2048,), jnp.int32),
        pltpu.VMEM((3 * SUBC,), jnp.int32),
        pltpu.VMEM((6 * SUBC,), jnp.float32),
        pltpu.SemaphoreType.DMA,
        pltpu.SemaphoreType.DMA,
        pltpu.SemaphoreType.DMA,
        pltpu.SemaphoreType.DMA,
    ],
    compiler_params=pltpu.CompilerParams(needs_layout_passes=False),
)
def _sc_project(points_hbm, colors_hbm, depth_hbm, img_hbm, idx_hbm,
                xb, yb, zb, depth_v, id_v, tmp_v, idx3, cstage,
                sem_pt, sem_cg, sem_im, sem_out):
    wid = lax.axis_index("s") * 2 + lax.axis_index("c")
    lane = lax.iota(jnp.int32, L)
    inf16 = jnp.full((L,), jnp.inf, dtype=jnp.float32)
    n16 = jnp.full((L,), N, dtype=jnp.int32)
    nc = N // CHUNK

    for k in range(NTASK // 32):
        t = wid + 32 * k
        b = t >> 3
        q = t & 7
        pbase = b * 4 * N
        cbase = b * 3 * N

        def init_body(i, _):
            depth_v[pl.ds(i * L, L)] = inf16
            id_v[pl.ds(i * L, L)] = n16
            return 0
        lax.fori_loop(0, R // L, init_body, 0)

        def fire_chunk(c, par):
            off = c * CHUNK
            pb = par * CHUNK
            pltpu.async_copy(points_hbm.at[pl.ds(pbase + off, CHUNK)],
                             xb.at[pl.ds(pb, CHUNK)], sem_pt)
            pltpu.async_copy(points_hbm.at[pl.ds(pbase + N + off, CHUNK)],
                             yb.at[pl.ds(pb, CHUNK)], sem_pt)
            pltpu.async_copy(points_hbm.at[pl.ds(pbase + 2 * N + off, CHUNK)],
                             zb.at[pl.ds(pb, CHUNK)], sem_pt)

        fire_chunk(0, 0)

        def chunk_body(c, _):
            par = c & 1
            off = c * CHUNK
            for ref in (xb, yb, zb):
                pltpu.make_async_copy(points_hbm.at[pl.ds(0, CHUNK)],
                                      ref.at[pl.ds(par * CHUNK, CHUNK)],
                                      sem_pt).wait()

            @pl.when(c + 1 < nc)
            def _():
                fire_chunk(c + 1, (c + 1) & 1)

            def process(jbase, lex, dirty):
                s = pl.ds(par * CHUNK + jbase, L)
                x = xb[s]
                y = yb[s]
                z = zb[s]
                u = jnp.minimum((x * jnp.float32(W)).astype(jnp.int32), W - 1)
                v = jnp.minimum((y * jnp.float32(H)).astype(jnp.int32), H - 1)
                pix = (v << 9) | u
                in_reg = (pix >> 15) == q
                local = pix & (R - 1)
                ids = (off + jbase) + lane
                d0 = plsc.load_gather(depth_v, [local])
                if lex:
                    i0 = plsc.load_gather(id_v, [local])
                    want = in_reg & ((z < d0) | ((z == d0) & (ids < i0)))
                else:
                    want = in_reg & (z < d0)
                slot = local & 2047
                plsc.store_scatter(tmp_v, [slot], lane, mask=want)
                winner = plsc.load_gather(tmp_v, [slot])
                lead = want & (winner == lane)
                plsc.store_scatter(depth_v, [local], z, mask=lead)
                plsc.store_scatter(id_v, [local], ids, mask=lead)
                return dirty | (want ^ lead)

            def make_pass(lex):
                def vec_body(j, dirty):
                    dirty = process(j * 2 * L, lex, dirty)
                    dirty = process(j * 2 * L + L, lex, dirty)
                    return dirty
                return lax.fori_loop(0, CHUNK // (2 * L), vec_body,
                                     jnp.zeros((L,), dtype=jnp.bool_))

            dirty = make_pass(lex=False)
            lax.while_loop(jnp.any, lambda d: make_pass(lex=True), dirty)
            return 0
        lax.fori_loop(0, nc, chunk_body, 0)

        def fin_body(i, _):
            s = pl.ds(i * L, L)
            idv = id_v[s]
            dv = depth_v[s]
            valid = idv < N
            id_v[s] = jnp.where(valid, idv, -1)
            depth_v[s] = jnp.where(valid, dv, 0.0)
            return 0
        lax.fori_loop(0, R // L, fin_body, 0)

        obase = (b * NREG + q) * R
        cp_d = pltpu.async_copy(depth_v, depth_hbm.at[pl.ds(obase, R)],
                                sem_out)
        cp_i = pltpu.async_copy(id_v, idx_hbm.at[pl.ds(obase, R)], sem_out)

        ibase0 = (b * 3 * NREG + q) * R

        def sub_body(sc_i, _):
            sbase = sc_i * SUBC
            cpar = sc_i & 1

            def bld(i, _):
                idv = id_v[pl.ds(sbase + i * L, L)]
                g = jnp.where(idv >= 0, idv + cbase, ZSLOT)
                idx3[pl.ds(i * L, L)] = g
                idx3[pl.ds(SUBC + i * L, L)] = g + N
                idx3[pl.ds(2 * SUBC + i * L, L)] = g + 2 * N
                return 0
            lax.fori_loop(0, SUBC // L, bld, 0)

            @pl.when(sc_i > 0)
            def _():
                for ch in range(3):
                    pltpu.make_async_copy(
                        cstage.at[pl.ds((1 - cpar) * 3 * SUBC + ch * SUBC,
                                        SUBC)],
                        img_hbm.at[pl.ds(0, SUBC)], sem_im).wait()

            cps = [pltpu.async_copy(
                       colors_hbm.at[idx3.at[pl.ds(ch * SUBC, SUBC)]],
                       cstage.at[pl.ds(cpar * 3 * SUBC + ch * SUBC, SUBC)],
                       sem_cg)
                   for ch in range(3)]
            for cp in cps:
                cp.wait()
            for ch in range(3):
                pltpu.async_copy(
                    cstage.at[pl.ds(cpar * 3 * SUBC + ch * SUBC, SUBC)],
                    img_hbm.at[pl.ds(ibase0 + ch * NREG * R + sbase, SUBC)],
                    sem_im)
            return 0
        lax.fori_loop(0, R // SUBC, sub_body, 0)

        last = (R // SUBC - 1) & 1
        for ch in range(3):
            pltpu.make_async_copy(
                cstage.at[pl.ds(last * 3 * SUBC + ch * SUBC, SUBC)],
                img_hbm.at[pl.ds(0, SUBC)], sem_im).wait()
        cp_d.wait()
        cp_i.wait()


def kernel(points, colors):
    cflat = jnp.pad(colors.reshape(-1), (0, 2 * N + 8))
    depth, img, index = _sc_project(points.reshape(-1), cflat)
    return (depth.reshape(B, H, W),
            img.reshape(B, 3, H, W),
            index.reshape(B, H, W))
